# Initial kernel scaffold; baseline (speedup 1.0000x reference)
import jax
import jax.numpy as jnp
from jax.experimental import pallas as pl


def kernel(x, edge_index, W_feat, b_feat, W_hyper, W_vis, b_vis, W_txt, b_txt):
    raise NotImplementedError("write your pallas kernel here")


# speedup vs baseline: 10.1960x; 10.1960x over previous
import functools

import jax
import jax.numpy as jnp
from jax import lax
from jax.experimental import pallas as pl
from jax.experimental.pallas import tpu as pltpu
from jax.experimental.pallas import tpu_sc as plsc

_N = 10000
_E = 320000
_HID = 64
_HYP = 32
_TAU = 0.5
_ALPHA = 0.1

_NC = 2
_NS = 16
_NW = _NC * _NS
_EPT = _E // _NW
_CH = 80
_NCH = _EPT // _CH
_RPT = _N // _NS
_LANES = 16


def _sc_deg_body(dst_hbm, out_hbm, idx_v, deg_v):
    c = lax.axis_index("c")
    s = lax.axis_index("s")
    wid = c * _NS + s
    zeros = jnp.zeros((_LANES,), jnp.float32)

    def zbody(i, carry):
        deg_v[pl.ds(i * _LANES, _LANES)] = zeros
        return carry

    lax.fori_loop(0, _N // _LANES, zbody, 0)
    pltpu.sync_copy(dst_hbm.at[wid], idx_v)
    ones = jnp.ones((_LANES,), jnp.float32)

    def body(i, carry):
        idx = idx_v[pl.ds(i * _LANES, _LANES)]
        plsc.addupdate_scatter(deg_v, [idx], ones)
        return carry

    lax.fori_loop(0, _EPT // _LANES, body, 0)
    pltpu.sync_copy(deg_v, out_hbm.at[wid])


_sc_deg = pl.kernel(
    _sc_deg_body,
    out_type=jax.ShapeDtypeStruct((_NW, _N), jnp.float32),
    mesh=plsc.VectorSubcoreMesh(core_axis_name="c", subcore_axis_name="s"),
    compiler_params=pltpu.CompilerParams(needs_layout_passes=False),
    scratch_types=[
        pltpu.VMEM((_EPT,), jnp.int32),
        pltpu.VMEM((_N,), jnp.float32),
    ],
)


def _sc_scat_body(z_hbm, src_hbm, dst_hbm, zero_hbm, out_hbm,
                  sidx_v, didx_v, rows_v, acc_sh, sem):
    c = lax.axis_index("c")
    s = lax.axis_index("s")
    wid = c * _NS + s
    pltpu.sync_copy(zero_hbm, acc_sh.at[pl.ds(s * _RPT, _RPT)])
    plsc.subcore_barrier()

    def body(j, carry):
        pltpu.sync_copy(src_hbm.at[wid, j], sidx_v)
        pltpu.sync_copy(dst_hbm.at[wid, j], didx_v)
        pltpu.async_copy(z_hbm.at[sidx_v], rows_v, sem).wait()
        pltpu.sync_copy(rows_v, acc_sh.at[didx_v], add=True)
        return carry

    lax.fori_loop(0, _NCH, body, 0)
    plsc.subcore_barrier()
    pltpu.sync_copy(acc_sh.at[pl.ds(s * _RPT, _RPT)],
                    out_hbm.at[c, pl.ds(s * _RPT, _RPT)])


_sc_scat = pl.kernel(
    _sc_scat_body,
    out_type=jax.ShapeDtypeStruct((_NC, _N, _HID), jnp.float32),
    mesh=plsc.VectorSubcoreMesh(core_axis_name="c", subcore_axis_name="s"),
    compiler_params=pltpu.CompilerParams(use_tc_tiling_on_sc=False),
    scratch_types=[
        pltpu.VMEM((_CH,), jnp.int32),
        pltpu.VMEM((_CH,), jnp.int32),
        pltpu.VMEM((_CH, _HID), jnp.float32),
        pltpu.VMEM_SHARED((_N, _HID), jnp.float32),
        pltpu.SemaphoreType.DMA,
    ],
)


def _tc_enc_body(x_ref, wf_ref, bf_ref, degp_ref, xemb_ref, z1_ref, dinv_ref):
    xe = jnp.dot(x_ref[...], wf_ref[...],
                 preferred_element_type=jnp.float32) + bf_ref[...][None, :]
    deg = jnp.sum(degp_ref[...], axis=0)
    dinv = jnp.where(deg > 0, lax.rsqrt(deg), 0.0)
    xemb_ref[...] = xe
    dinv_ref[...] = dinv
    z1_ref[...] = xe * dinv[:, None]


def _tc_comb_body(p_ref, dinv_ref, accp_ref, acc_ref, z_ref):
    dinv = dinv_ref[...]
    cur = (p_ref[0] + p_ref[1]) * dinv[:, None]
    acc_ref[...] = accp_ref[...] + cur
    z_ref[...] = cur * dinv[:, None]


def _tc_fin_body(p_ref, dinv_ref, accp_ref, xemb_ref, g_ref, wh_ref,
                 wv_ref, bv_ref, wt_ref, bt_ref,
                 fin_ref, vis_ref, txt_ref):
    dinv = dinv_ref[...]
    c3 = (p_ref[0] + p_ref[1]) * dinv[:, None]
    local = (accp_ref[...] + c3) * 0.25
    xe = xemb_ref[...]
    logits = (jnp.dot(xe, wh_ref[...], preferred_element_type=jnp.float32)
              + g_ref[...]) * (1.0 / _TAU)
    m = jnp.max(logits, axis=1, keepdims=True)
    e = jnp.exp(logits - m)
    h = e / jnp.sum(e, axis=1, keepdims=True)
    lat = lax.dot_general(h, xe, (((0,), (0,)), ((), ())),
                          preferred_element_type=jnp.float32)
    glob = jnp.dot(h, lat, preferred_element_type=jnp.float32)
    nrm = jnp.sqrt(jnp.sum(glob * glob, axis=1, keepdims=True))
    gn = glob / jnp.maximum(nrm, 1e-12)
    fin = local + _ALPHA * gn
    fin_ref[...] = fin
    vis_ref[...] = jnp.maximum(
        jnp.dot(fin, wv_ref[...], preferred_element_type=jnp.float32)
        + bv_ref[...][None, :], 0.0)
    txt_ref[...] = jnp.maximum(
        jnp.dot(fin, wt_ref[...], preferred_element_type=jnp.float32)
        + bt_ref[...][None, :], 0.0)


_f32 = jnp.float32

_tc_enc = pl.pallas_call(
    _tc_enc_body,
    out_shape=(jax.ShapeDtypeStruct((_N, _HID), _f32),
               jax.ShapeDtypeStruct((_N, _HID), _f32),
               jax.ShapeDtypeStruct((_N,), _f32)),
)

_tc_comb = pl.pallas_call(
    _tc_comb_body,
    out_shape=(jax.ShapeDtypeStruct((_N, _HID), _f32),
               jax.ShapeDtypeStruct((_N, _HID), _f32)),
)

_tc_fin = pl.pallas_call(
    _tc_fin_body,
    out_shape=(jax.ShapeDtypeStruct((_N, _HID), _f32),
               jax.ShapeDtypeStruct((_N, _HID), _f32),
               jax.ShapeDtypeStruct((_N, _HID), _f32)),
)


def kernel(x, edge_index, W_feat, b_feat, W_hyper, W_vis, b_vis, W_txt, b_txt):
    src3 = edge_index[0].astype(jnp.int32).reshape(_NW, _NCH, _CH)
    dst = edge_index[1].astype(jnp.int32)
    dst2 = dst.reshape(_NW, _EPT)
    dst3 = dst.reshape(_NW, _NCH, _CH)
    zeros_tile = jnp.zeros((_RPT, _HID), _f32)
    g = jax.random.gumbel(jax.random.key(1), (_N, _HYP), _f32)

    degp = _sc_deg(dst2)
    xemb, z1, dinv = _tc_enc(x, W_feat, b_feat, degp)
    p1 = _sc_scat(z1, src3, dst3, zeros_tile)
    acc1, z2 = _tc_comb(p1, dinv, xemb)
    p2 = _sc_scat(z2, src3, dst3, zeros_tile)
    acc2, z3 = _tc_comb(p2, dinv, acc1)
    p3 = _sc_scat(z3, src3, dst3, zeros_tile)
    return _tc_fin(p3, dinv, acc2, xemb, g, W_hyper, W_vis, b_vis, W_txt, b_txt)
